# baseline (speedup 1.0000x reference)
import functools

import jax
import jax.numpy as jnp
from jax import lax
from jax.experimental import pallas as pl
from jax.experimental.pallas import tpu as pltpu
from jax.experimental.pallas import tpu_sc as plsc

_EMB = 1000000
_DIM = 32
_BATCH = 16384

_NC = 2
_NS = 16
_NW = _NC * _NS
_BPW = _BATCH // _NW


def _gather_kernel(idx_hbm, table_hbm, out_hbm, idx_v, rows_v, sem):
    wid = lax.axis_index("s") * _NC + lax.axis_index("c")
    base = wid * _BPW
    pltpu.sync_copy(idx_hbm.at[pl.ds(base, _BPW)], idx_v)
    pltpu.async_copy(table_hbm.at[idx_v], rows_v, sem).wait()
    pltpu.sync_copy(rows_v, out_hbm.at[pl.ds(base, _BPW)])


@jax.jit
def _gather(idx, table):
    mesh = plsc.VectorSubcoreMesh(core_axis_name="c", subcore_axis_name="s")
    return pl.kernel(
        _gather_kernel,
        mesh=mesh,
        out_type=jax.ShapeDtypeStruct((_BATCH, _DIM), jnp.float32),
        scratch_types=[
            pltpu.VMEM((_BPW,), jnp.int32),
            pltpu.VMEM((_BPW, _DIM), jnp.float32),
            pltpu.SemaphoreType.DMA,
        ],
        compiler_params=pltpu.CompilerParams(use_tc_tiling_on_sc=False),
    )(idx, table)


def kernel(input, weight):
    return _gather(input.astype(jnp.int32), weight)


# speedup vs baseline: 3.5797x; 3.5797x over previous
import jax
import jax.numpy as jnp
from jax import lax
from jax.experimental import pallas as pl
from jax.experimental.pallas import tpu as pltpu
from jax.experimental.pallas import tpu_sc as plsc

_EMB = 1000000
_DIM = 32
_BATCH = 16384

_NC = 2
_NS = 16
_NW = _NC * _NS
_BPW = _BATCH // _NW
_GRP = 16
_LAST_TILE = (_EMB // 128) * 128
_TAIL = _EMB - _LAST_TILE


def _gather_kernel(idx_hbm, wt_hbm, out_hbm, idx_v, buf_v, col_v, sem):
    wid = lax.axis_index("s") * _NC + lax.axis_index("c")
    base = wid * _BPW
    pltpu.sync_copy(idx_hbm.at[pl.ds(base, _BPW)], idx_v)

    c_lo = lax.iota(jnp.int32, 16)
    c_hi = c_lo + 16

    def body(k, _):
        rv = idx_v[pl.ds(k * _GRP, _GRP)]
        for i in range(_GRP):
            r = rv[i]
            rb = pl.multiple_of((r >> 7) << 7, 128)
            pltpu.async_copy(
                wt_hbm.at[:, pl.ds(rb, 128)],
                buf_v.at[:, pl.ds(i * 128, 128)],
                sem,
            )

        pltpu.make_async_copy(
            wt_hbm.at[:, pl.ds(0, _GRP * 128)], buf_v, sem
        ).wait()

        for i in range(_GRP):
            r = rv[i]
            rr = i * 128 + (r & 127)
            rr_v = jnp.full((16,), rr, jnp.int32)
            j_v = jnp.full((16,), k * _GRP + i, jnp.int32)
            v0 = plsc.load_gather(buf_v, [c_lo, rr_v])
            v1 = plsc.load_gather(buf_v, [c_hi, rr_v])
            plsc.store_scatter(col_v, [c_lo, j_v], v0)
            plsc.store_scatter(col_v, [c_hi, j_v], v1)
        return 0

    lax.fori_loop(0, _BPW // _GRP, body, 0)
    pltpu.sync_copy(col_v, out_hbm.at[:, pl.ds(base, _BPW)])


@jax.jit
def _gather(idx, w_t):
    mesh = plsc.VectorSubcoreMesh(core_axis_name="c", subcore_axis_name="s")
    return pl.kernel(
        _gather_kernel,
        mesh=mesh,
        out_type=jax.ShapeDtypeStruct((_DIM, _BATCH), jnp.float32),
        scratch_types=[
            pltpu.VMEM((_BPW,), jnp.int32),
            pltpu.VMEM((_DIM, _GRP * 128), jnp.float32),
            pltpu.VMEM((_DIM, _BPW), jnp.float32),
            pltpu.SemaphoreType.DMA,
        ],
        compiler_params=pltpu.CompilerParams(needs_layout_passes=False),
    )(idx, w_t)


def kernel(input, weight):
    out_t = _gather(input.astype(jnp.int32), weight.T)
    return out_t.T
